# Initial kernel scaffold; baseline (speedup 1.0000x reference)
import jax
import jax.numpy as jnp
from jax.experimental import pallas as pl


def kernel(entity_embs, relation_embs, tuples, node_indices, edge_indices, W1, W2, a1, a2):
    raise NotImplementedError("write your pallas kernel here")


# speedup vs baseline: 3.6123x; 3.6123x over previous
import jax
import jax.numpy as jnp
import numpy as np
from jax import lax
from jax.experimental import pallas as pl
from jax.experimental.pallas import tpu as pltpu
from jax.experimental.pallas import tpu_sc as plsc

_N = 10000
_R = 10000
_M = 20000
_NNZ = 640000
_D = 128
_AR = 6

_NC = 2
_NS = 16
_NW = _NC * _NS

_MP = 20480
_TUP_PER = _MP // _NW
_CHUNKS = _TUP_PER // 16

_NNZP = 655360
_NNZ_PER = _NNZP // _NS
_BLK_NNZ = 128
_NBLK = _NNZ_PER // _BLK_NNZ
_NH = _N // _NC
_ACC_ROWS = 6000

_NEG = np.float32(-9e15)
_SLOPE = np.float32(0.2)
_ONE = np.float32(1.0)
_BIG = np.float32(1e30)


def _dense_body(ent_ref, rel_ref, w1_ref, w2_ref, a1_ref, a2_ref,
                entw_ref, s1_ref, t2_ref, s2_ref):
    ew = jnp.dot(ent_ref[...], w1_ref[...], preferred_element_type=jnp.float32)
    entw_ref[...] = ew
    a1 = a1_ref[...]
    a2 = a2_ref[...]
    s1_ref[...] = jnp.dot(ew, a1[:_D, :], preferred_element_type=jnp.float32)
    t2_ref[...] = jnp.dot(ew, a2[_D:, :], preferred_element_type=jnp.float32)
    b2 = jnp.dot(w2_ref[...], a1[_D:, :], preferred_element_type=jnp.float32)
    s2_ref[...] = jnp.dot(rel_ref[...], b2, preferred_element_type=jnp.float32)


def _dense_call(entity_embs, relation_embs, w1, w2, a1, a2):
    blk = 2000
    grid = (_N // blk,)
    return pl.pallas_call(
        _dense_body,
        grid=grid,
        in_specs=[
            pl.BlockSpec((blk, _D), lambda i: (i, 0)),
            pl.BlockSpec((blk, _D), lambda i: (i, 0)),
            pl.BlockSpec((_D, _D), lambda i: (0, 0)),
            pl.BlockSpec((_D, _D), lambda i: (0, 0)),
            pl.BlockSpec((2 * _D, 1), lambda i: (0, 0)),
            pl.BlockSpec((2 * _D, 1), lambda i: (0, 0)),
        ],
        out_specs=[
            pl.BlockSpec((blk, _D), lambda i: (i, 0)),
            pl.BlockSpec((blk, 1), lambda i: (i, 0)),
            pl.BlockSpec((blk, 1), lambda i: (i, 0)),
            pl.BlockSpec((blk, 1), lambda i: (i, 0)),
        ],
        out_shape=[
            jax.ShapeDtypeStruct((_N, _D), jnp.float32),
            jax.ShapeDtypeStruct((_N, 1), jnp.float32),
            jax.ShapeDtypeStruct((_N, 1), jnp.float32),
            jax.ShapeDtypeStruct((_R, 1), jnp.float32),
        ],
    )(entity_embs, relation_embs, w1, w2, a1, a2)


def _sc_edge_body(tup_hbm, s1_hbm, s2_hbm, entw_hbm, edge_hbm,
                  tup_v, s1_v, s2_v, eidx_v, rows_v, out_v, sem):
    c = lax.axis_index("c")
    s = lax.axis_index("s")
    wid = s * _NC + c
    base_row = wid * _TUP_PER
    pltpu.sync_copy(tup_hbm.at[pl.ds(base_row * (_AR + 1), _TUP_PER * (_AR + 1))],
                    tup_v)
    pltpu.sync_copy(s1_hbm, s1_v)
    pltpu.sync_copy(s2_hbm, s2_v)

    def chunk(k, carry):
        rowids = (k * 16 + lax.iota(jnp.int32, 16)) * (_AR + 1)
        rel = plsc.load_gather(tup_v, [rowids])
        ridx = rel - 1
        ridx = jnp.where(ridx < 0, ridx + _R, ridx)
        s2g = plsc.load_gather(s2_v, [ridx])
        logits = []
        for ar in range(_AR):
            t = plsc.load_gather(tup_v, [rowids + (ar + 1)])
            eidx = t - 1
            eidx = jnp.where(eidx < 0, eidx + _N, eidx)
            eidx_v[pl.ds(ar * 16, 16)] = eidx
            lg = plsc.load_gather(s1_v, [eidx])
            lg = jnp.where(t > 0, lg, _NEG)
            x = lg + s2g
            logits.append(jnp.where(x > 0, x, _SLOPE * x))
        mx = logits[0]
        for ar in range(1, _AR):
            mx = jnp.maximum(mx, logits[ar])
        es = [jnp.exp(l - mx) for l in logits]
        tot = es[0]
        for ar in range(1, _AR):
            tot = tot + es[ar]
        inv = _ONE / tot
        att = [es[ar] * inv for ar in range(_AR)]
        pltpu.async_copy(entw_hbm.at[eidx_v], rows_v, sem).wait()
        for ti in range(16):
            for j in range(8):
                acc = att[0][ti] * rows_v[0 * 16 + ti, pl.ds(j * 16, 16)]
                for ar in range(1, _AR):
                    acc = acc + att[ar][ti] * rows_v[ar * 16 + ti, pl.ds(j * 16, 16)]
                out_v[ti, pl.ds(j * 16, 16)] = jnp.where(
                    acc > 0, acc, jnp.exp(acc) - _ONE)
        pltpu.sync_copy(out_v, edge_hbm.at[pl.ds(base_row + k * 16, 16), :])
        return carry

    lax.fori_loop(0, _CHUNKS, chunk, 0)


def _sc_edge_call(tup_pad, s1v, s2v, entity_w):
    mesh = plsc.VectorSubcoreMesh(core_axis_name="c", subcore_axis_name="s")
    fn = pl.kernel(
        _sc_edge_body,
        out_type=jax.ShapeDtypeStruct((_MP, _D), jnp.float32),
        mesh=mesh,
        compiler_params=pltpu.CompilerParams(needs_layout_passes=False),
        scratch_types=[
            pltpu.VMEM((_TUP_PER * (_AR + 1),), jnp.int32),
            pltpu.VMEM((_N,), jnp.float32),
            pltpu.VMEM((_R,), jnp.float32),
            pltpu.VMEM((_AR * 16,), jnp.int32),
            pltpu.VMEM((_AR * 16, _D), jnp.float32),
            pltpu.VMEM((16, _D), jnp.float32),
            pltpu.SemaphoreType.DMA,
        ],
    )
    return fn(tup_pad, s1v, s2v, entity_w)


def _t1_body(ed_ref, a2_ref, t1_ref):
    blk = ed_ref.shape[0]
    a2 = a2_ref[...]
    v = jnp.dot(ed_ref[...], a2[:_D, :], preferred_element_type=jnp.float32)
    row = (pl.program_id(0) * blk
           + lax.broadcasted_iota(jnp.int32, (blk, 1), 0))
    t1_ref[...] = jnp.where(row < _M, v, _BIG)


def _t1_call(edge_embs, a2):
    blk = 2048
    nb = _MP // blk
    return pl.pallas_call(
        _t1_body,
        grid=(nb,),
        in_specs=[
            pl.BlockSpec((blk, _D), lambda i: (i, 0)),
            pl.BlockSpec((2 * _D, 1), lambda i: (0, 0)),
        ],
        out_specs=pl.BlockSpec((blk, 1), lambda i: (i, 0)),
        out_shape=jax.ShapeDtypeStruct((_MP, 1), jnp.float32),
    )(edge_embs, a2)


def _sc_agg_body(nidx_hbm, eidx_hbm, t1_hbm, t2_hbm, edge_hbm, zero_hbm,
                 acc_out, rsum_out,
                 t1_v, t2_v, rsum_v, nidx_v, nloc_v, eidx_v, rows_v, acc_sh,
                 sem):
    c = lax.axis_index("c")
    s = lax.axis_index("s")

    @pl.when(s == 0)
    def _zero_acc():
        pltpu.sync_copy(zero_hbm, acc_sh)

    pltpu.sync_copy(t1_hbm, t1_v)
    pltpu.sync_copy(t2_hbm, t2_v)
    z16 = jnp.zeros((16,), jnp.float32)

    def zloop(i, carry):
        rsum_v[pl.ds(i * 16, 16)] = z16
        return carry

    lax.fori_loop(0, _N // 16, zloop, 0)
    plsc.subcore_barrier()

    base = s * _NNZ_PER
    nshift = c * _NH

    def blk(j, carry):
        off = base + j * _BLK_NNZ
        pltpu.sync_copy(nidx_hbm.at[pl.ds(off, _BLK_NNZ)], nidx_v.at[0])
        pltpu.sync_copy(eidx_hbm.at[pl.ds(off, _BLK_NNZ)], eidx_v.at[0])
        evecs = []
        for k in range(_BLK_NNZ // 16):
            ni = nidx_v[0, pl.ds(k * 16, 16)]
            ei = eidx_v[0, pl.ds(k * 16, 16)]
            x = plsc.load_gather(t1_v, [ei]) + plsc.load_gather(t2_v, [ni])
            x = jnp.where(x > 0, x, _SLOPE * x)
            e = jnp.exp(-x)
            evecs.append(e)
            plsc.addupdate_scatter(rsum_v, [ni], e)
            nl = ni - nshift
            nl = jnp.where((nl >= 0) & (nl < _NH), nl, _NH)
            nloc_v[0, pl.ds(k * 16, 16)] = nl
        pltpu.async_copy(edge_hbm.at[eidx_v.at[0]], rows_v, sem).wait()
        for b in range(_BLK_NNZ):
            w = evecs[b // 16][b % 16]
            for jj in range(_D // 16):
                sl = pl.ds(jj * 16, 16)
                rows_v[b, sl] = rows_v[b, sl] * w
        pltpu.sync_copy(rows_v, acc_sh.at[nloc_v.at[0]], add=True)
        return carry

    lax.fori_loop(0, _NBLK, blk, 0)
    plsc.subcore_barrier()

    @pl.when(s == 0)
    def _write_acc():
        pltpu.sync_copy(acc_sh, acc_out.at[c])

    @pl.when(c == 0)
    def _write_rsum():
        pltpu.sync_copy(rsum_v, rsum_out.at[s])


def _sc_agg_call(nidx_pad, eidx_pad, t1v, t2v, edge_embs, zeros_init):
    mesh = plsc.VectorSubcoreMesh(core_axis_name="c", subcore_axis_name="s")
    fn = pl.kernel(
        _sc_agg_body,
        out_type=[
            jax.ShapeDtypeStruct((_NC, _ACC_ROWS, _D), jnp.float32),
            jax.ShapeDtypeStruct((_NS, _N), jnp.float32),
        ],
        mesh=mesh,
        compiler_params=pltpu.CompilerParams(needs_layout_passes=False),
        scratch_types=[
            pltpu.VMEM((_MP,), jnp.float32),
            pltpu.VMEM((_N,), jnp.float32),
            pltpu.VMEM((_N,), jnp.float32),
            pltpu.VMEM((1, _BLK_NNZ), jnp.int32),
            pltpu.VMEM((1, _BLK_NNZ), jnp.int32),
            pltpu.VMEM((1, _BLK_NNZ), jnp.int32),
            pltpu.VMEM((_BLK_NNZ, _D), jnp.float32),
            pltpu.VMEM_SHARED((_ACC_ROWS, _D), jnp.float32),
            pltpu.SemaphoreType.DMA,
        ],
    )
    return fn(nidx_pad, eidx_pad, t1v, t2v, edge_embs, zeros_init)


def _final_body(acc_ref, rsum_ref, out_ref):
    num = acc_ref[...]
    rs = jnp.sum(rsum_ref[...], axis=1)
    x = num / rs[:, None]
    out_ref[...] = jnp.where(x > 0, x, jnp.exp(x) - _ONE)


def _final_call(acc_flat, rsum_t):
    blk = 1000
    nb_core = _ACC_ROWS // blk
    nreal = _NH // blk

    def acc_map(i):
        return (jnp.where(i < nreal, i, i + (nb_core - nreal)), 0)

    return pl.pallas_call(
        _final_body,
        grid=(_N // blk,),
        in_specs=[
            pl.BlockSpec((blk, _D), acc_map),
            pl.BlockSpec((blk, _NS), lambda i: (i, 0)),
        ],
        out_specs=pl.BlockSpec((blk, _D), lambda i: (i, 0)),
        out_shape=jax.ShapeDtypeStruct((_N, _D), jnp.float32),
    )(acc_flat, rsum_t)


def kernel(entity_embs, relation_embs, tuples, node_indices, edge_indices,
           W1, W2, a1, a2):
    entity_w, s1, t2, s2 = _dense_call(entity_embs, relation_embs, W1, W2, a1, a2)
    s1v = s1[:, 0]
    t2v = t2[:, 0]
    s2v = s2[:, 0]

    tup_pad = jnp.concatenate(
        [tuples.astype(jnp.int32),
         jnp.zeros((_MP - _M, _AR + 1), jnp.int32)], axis=0)
    edge_embs = _sc_edge_call(tup_pad.reshape(-1), s1v, s2v, entity_w)

    t1 = _t1_call(edge_embs, a2)
    t1v = t1[:, 0]

    nidx_pad = jnp.concatenate(
        [node_indices.astype(jnp.int32),
         jnp.zeros((_NNZP - _NNZ,), jnp.int32)])
    eidx_pad = jnp.concatenate(
        [edge_indices.astype(jnp.int32),
         jnp.full((_NNZP - _NNZ,), _M, jnp.int32)])
    zeros_init = jnp.zeros((_ACC_ROWS, _D), jnp.float32)

    acc, rsum = _sc_agg_call(nidx_pad, eidx_pad, t1v, t2v, edge_embs,
                             zeros_init)
    return _final_call(acc.reshape(_NC * _ACC_ROWS, _D), rsum.T)


# speedup vs baseline: 4.6491x; 1.2870x over previous
import jax
import jax.numpy as jnp
import numpy as np
from jax import lax
from jax.experimental import pallas as pl
from jax.experimental.pallas import tpu as pltpu
from jax.experimental.pallas import tpu_sc as plsc

_N = 10000
_R = 10000
_M = 20000
_NNZ = 640000
_D = 128
_AR = 6

_NC = 2
_NS = 16
_NW = _NC * _NS

_MP = 20480
_TUP_PER = _MP // _NW
_CHUNKS = _TUP_PER // 16

_NNZP = 655360
_NNZ_PER = _NNZP // _NW
_BLK_NNZ = 64
_NBLK = _NNZ_PER // _BLK_NNZ

_NEG = np.float32(-9e15)
_SLOPE = np.float32(0.2)
_ONE = np.float32(1.0)
_BIG = np.float32(1e30)


def _dense_body(ent_ref, rel_ref, w1_ref, w2_ref, a1_ref, a2_ref,
                entw_ref, s1_ref, t2_ref, s2_ref):
    ew = jnp.dot(ent_ref[...], w1_ref[...], preferred_element_type=jnp.float32)
    entw_ref[...] = ew
    a1 = a1_ref[...]
    a2 = a2_ref[...]
    s1_ref[...] = jnp.dot(ew, a1[:_D, :], preferred_element_type=jnp.float32)
    t2_ref[...] = jnp.dot(ew, a2[_D:, :], preferred_element_type=jnp.float32)
    b2 = jnp.dot(w2_ref[...], a1[_D:, :], preferred_element_type=jnp.float32)
    s2_ref[...] = jnp.dot(rel_ref[...], b2, preferred_element_type=jnp.float32)


def _dense_call(entity_embs, relation_embs, w1, w2, a1, a2):
    blk = 2000
    grid = (_N // blk,)
    return pl.pallas_call(
        _dense_body,
        grid=grid,
        in_specs=[
            pl.BlockSpec((blk, _D), lambda i: (i, 0)),
            pl.BlockSpec((blk, _D), lambda i: (i, 0)),
            pl.BlockSpec((_D, _D), lambda i: (0, 0)),
            pl.BlockSpec((_D, _D), lambda i: (0, 0)),
            pl.BlockSpec((2 * _D, 1), lambda i: (0, 0)),
            pl.BlockSpec((2 * _D, 1), lambda i: (0, 0)),
        ],
        out_specs=[
            pl.BlockSpec((blk, _D), lambda i: (i, 0)),
            pl.BlockSpec((blk, 1), lambda i: (i, 0)),
            pl.BlockSpec((blk, 1), lambda i: (i, 0)),
            pl.BlockSpec((blk, 1), lambda i: (i, 0)),
        ],
        out_shape=[
            jax.ShapeDtypeStruct((_N, _D), jnp.float32),
            jax.ShapeDtypeStruct((_N, 1), jnp.float32),
            jax.ShapeDtypeStruct((_N, 1), jnp.float32),
            jax.ShapeDtypeStruct((_R, 1), jnp.float32),
        ],
    )(entity_embs, relation_embs, w1, w2, a1, a2)


def _sc_edge_body(tup_hbm, s1_hbm, s2_hbm, entw_hbm, edge_hbm,
                  tup_v, s1_v, s2_v, eidx_v, rows_v, out_v, sem):
    c = lax.axis_index("c")
    s = lax.axis_index("s")
    wid = s * _NC + c
    base_row = wid * _TUP_PER
    pltpu.sync_copy(tup_hbm.at[pl.ds(base_row * (_AR + 1), _TUP_PER * (_AR + 1))],
                    tup_v)
    pltpu.sync_copy(s1_hbm, s1_v)
    pltpu.sync_copy(s2_hbm, s2_v)

    def chunk(k, carry):
        rowids = (k * 16 + lax.iota(jnp.int32, 16)) * (_AR + 1)
        rel = plsc.load_gather(tup_v, [rowids])
        ridx = rel - 1
        ridx = jnp.where(ridx < 0, ridx + _R, ridx)
        s2g = plsc.load_gather(s2_v, [ridx])
        logits = []
        for ar in range(_AR):
            t = plsc.load_gather(tup_v, [rowids + (ar + 1)])
            eidx = t - 1
            eidx = jnp.where(eidx < 0, eidx + _N, eidx)
            eidx_v[pl.ds(ar * 16, 16)] = eidx
            lg = plsc.load_gather(s1_v, [eidx])
            lg = jnp.where(t > 0, lg, _NEG)
            x = lg + s2g
            logits.append(jnp.where(x > 0, x, _SLOPE * x))
        mx = logits[0]
        for ar in range(1, _AR):
            mx = jnp.maximum(mx, logits[ar])
        es = [jnp.exp(l - mx) for l in logits]
        tot = es[0]
        for ar in range(1, _AR):
            tot = tot + es[ar]
        inv = _ONE / tot
        att = [es[ar] * inv for ar in range(_AR)]
        pltpu.async_copy(entw_hbm.at[eidx_v], rows_v, sem).wait()
        for ti in range(16):
            for j in range(8):
                acc = att[0][ti] * rows_v[0 * 16 + ti, pl.ds(j * 16, 16)]
                for ar in range(1, _AR):
                    acc = acc + att[ar][ti] * rows_v[ar * 16 + ti, pl.ds(j * 16, 16)]
                out_v[ti, pl.ds(j * 16, 16)] = jnp.where(
                    acc > 0, acc, jnp.exp(acc) - _ONE)
        pltpu.sync_copy(out_v, edge_hbm.at[pl.ds(base_row + k * 16, 16), :])
        return carry

    lax.fori_loop(0, _CHUNKS, chunk, 0)


def _sc_edge_call(tup_pad, s1v, s2v, entity_w):
    mesh = plsc.VectorSubcoreMesh(core_axis_name="c", subcore_axis_name="s")
    fn = pl.kernel(
        _sc_edge_body,
        out_type=jax.ShapeDtypeStruct((_MP, _D), jnp.float32),
        mesh=mesh,
        compiler_params=pltpu.CompilerParams(needs_layout_passes=False),
        scratch_types=[
            pltpu.VMEM((_TUP_PER * (_AR + 1),), jnp.int32),
            pltpu.VMEM((_N,), jnp.float32),
            pltpu.VMEM((_R,), jnp.float32),
            pltpu.VMEM((_AR * 16,), jnp.int32),
            pltpu.VMEM((_AR * 16, _D), jnp.float32),
            pltpu.VMEM((16, _D), jnp.float32),
            pltpu.SemaphoreType.DMA,
        ],
    )
    return fn(tup_pad, s1v, s2v, entity_w)


def _t1_body(ed_ref, a2_ref, t1_ref):
    blk = ed_ref.shape[0]
    a2 = a2_ref[...]
    v = jnp.dot(ed_ref[...], a2[:_D, :], preferred_element_type=jnp.float32)
    row = (pl.program_id(0) * blk
           + lax.broadcasted_iota(jnp.int32, (blk, 1), 0))
    t1_ref[...] = jnp.where(row < _M, v, _BIG)


def _t1_call(edge_embs, a2):
    blk = 2048
    nb = _MP // blk
    return pl.pallas_call(
        _t1_body,
        grid=(nb,),
        in_specs=[
            pl.BlockSpec((blk, _D), lambda i: (i, 0)),
            pl.BlockSpec((2 * _D, 1), lambda i: (0, 0)),
        ],
        out_specs=pl.BlockSpec((blk, 1), lambda i: (i, 0)),
        out_shape=jax.ShapeDtypeStruct((_MP, 1), jnp.float32),
    )(edge_embs, a2)


def _sc_agg_body(nidx_hbm, eidx_hbm, t1_hbm, t2_hbm, edge_hbm, zero_hbm,
                 acc_out, rsum_out,
                 t1_v, t2_v, rsum_v, nidx_v, eidx_v, rows_v, acc_sh,
                 sem):
    c = lax.axis_index("c")
    s = lax.axis_index("s")
    wid = s * _NC + c

    @pl.when(s == 0)
    def _zero_acc():
        pltpu.sync_copy(zero_hbm, acc_sh)

    pltpu.sync_copy(t1_hbm, t1_v)
    pltpu.sync_copy(t2_hbm, t2_v)
    z16 = jnp.zeros((16,), jnp.float32)

    def zloop(i, carry):
        rsum_v[pl.ds(i * 16, 16)] = z16
        return carry

    lax.fori_loop(0, _N // 16, zloop, 0)
    plsc.subcore_barrier()

    base = wid * _NNZ_PER

    def blk(j, carry):
        off = base + j * _BLK_NNZ
        pltpu.sync_copy(nidx_hbm.at[pl.ds(off, _BLK_NNZ)], nidx_v.at[0])
        pltpu.sync_copy(eidx_hbm.at[pl.ds(off, _BLK_NNZ)], eidx_v.at[0])
        evecs = []
        for k in range(_BLK_NNZ // 16):
            ni = nidx_v[0, pl.ds(k * 16, 16)]
            ei = eidx_v[0, pl.ds(k * 16, 16)]
            x = plsc.load_gather(t1_v, [ei]) + plsc.load_gather(t2_v, [ni])
            x = jnp.where(x > 0, x, _SLOPE * x)
            e = jnp.exp(-x)
            evecs.append(e)
            plsc.addupdate_scatter(rsum_v, [ni], e)
        pltpu.async_copy(edge_hbm.at[eidx_v.at[0]], rows_v, sem).wait()
        for b in range(_BLK_NNZ):
            w = evecs[b // 16][b % 16]
            for jj in range(_D // 16):
                sl = pl.ds(jj * 16, 16)
                rows_v[b, sl] = rows_v[b, sl] * w
        pltpu.sync_copy(rows_v, acc_sh.at[nidx_v.at[0]], add=True)
        return carry

    lax.fori_loop(0, _NBLK, blk, 0)
    plsc.subcore_barrier()

    @pl.when(s == 0)
    def _write_acc():
        pltpu.sync_copy(acc_sh, acc_out.at[c])

    pltpu.sync_copy(rsum_v, rsum_out.at[wid])


def _sc_agg_call(nidx_pad, eidx_pad, t1v, t2v, edge_embs, zeros_init):
    mesh = plsc.VectorSubcoreMesh(core_axis_name="c", subcore_axis_name="s")
    fn = pl.kernel(
        _sc_agg_body,
        out_type=[
            jax.ShapeDtypeStruct((_NC, _N, _D), jnp.float32),
            jax.ShapeDtypeStruct((_NW, _N), jnp.float32),
        ],
        mesh=mesh,
        compiler_params=pltpu.CompilerParams(needs_layout_passes=False),
        scratch_types=[
            pltpu.VMEM((_MP,), jnp.float32),
            pltpu.VMEM((_N,), jnp.float32),
            pltpu.VMEM((_N,), jnp.float32),
            pltpu.VMEM((1, _BLK_NNZ), jnp.int32),
            pltpu.VMEM((1, _BLK_NNZ), jnp.int32),
            pltpu.VMEM((_BLK_NNZ, _D), jnp.float32),
            pltpu.VMEM_SHARED((_N, _D), jnp.float32),
            pltpu.SemaphoreType.DMA,
        ],
    )
    return fn(nidx_pad, eidx_pad, t1v, t2v, edge_embs, zeros_init)


def _final_body(acc0_ref, acc1_ref, rsum_ref, out_ref):
    num = acc0_ref[...] + acc1_ref[...]
    rs = jnp.sum(rsum_ref[...], axis=1)
    x = num / rs[:, None]
    out_ref[...] = jnp.where(x > 0, x, jnp.exp(x) - _ONE)


def _final_call(acc0, acc1, rsum_t):
    blk = 1000
    return pl.pallas_call(
        _final_body,
        grid=(_N // blk,),
        in_specs=[
            pl.BlockSpec((blk, _D), lambda i: (i, 0)),
            pl.BlockSpec((blk, _D), lambda i: (i, 0)),
            pl.BlockSpec((blk, _NW), lambda i: (i, 0)),
        ],
        out_specs=pl.BlockSpec((blk, _D), lambda i: (i, 0)),
        out_shape=jax.ShapeDtypeStruct((_N, _D), jnp.float32),
    )(acc0, acc1, rsum_t)


def kernel(entity_embs, relation_embs, tuples, node_indices, edge_indices,
           W1, W2, a1, a2):
    entity_w, s1, t2, s2 = _dense_call(entity_embs, relation_embs, W1, W2, a1, a2)
    s1v = s1[:, 0]
    t2v = t2[:, 0]
    s2v = s2[:, 0]

    tup_pad = jnp.concatenate(
        [tuples.astype(jnp.int32),
         jnp.zeros((_MP - _M, _AR + 1), jnp.int32)], axis=0)
    edge_embs = _sc_edge_call(tup_pad.reshape(-1), s1v, s2v, entity_w)

    t1 = _t1_call(edge_embs, a2)
    t1v = t1[:, 0]

    nidx_pad = jnp.concatenate(
        [node_indices.astype(jnp.int32),
         jnp.zeros((_NNZP - _NNZ,), jnp.int32)])
    eidx_pad = jnp.concatenate(
        [edge_indices.astype(jnp.int32),
         jnp.full((_NNZP - _NNZ,), _M, jnp.int32)])
    zeros_init = jnp.zeros((_N, _D), jnp.float32)

    acc, rsum = _sc_agg_call(nidx_pad, eidx_pad, t1v, t2v, edge_embs,
                             zeros_init)
    return _final_call(acc[0], acc[1], rsum.T)


# speedup vs baseline: 4.6762x; 1.0058x over previous
import jax
import jax.numpy as jnp
import numpy as np
from jax import lax
from jax.experimental import pallas as pl
from jax.experimental.pallas import tpu as pltpu
from jax.experimental.pallas import tpu_sc as plsc

_N = 10000
_R = 10000
_M = 20000
_NNZ = 640000
_D = 128
_AR = 6

_NC = 2
_NS = 16
_NW = _NC * _NS

_MP = 20480
_TUP_PER = _MP // _NW
_CHUNKS = _TUP_PER // 16

_NNZP = 655360
_NNZ_PER = _NNZP // _NW
_BLK_NNZ = 64
_NBLK = _NNZ_PER // _BLK_NNZ

_NEG = np.float32(-9e15)
_SLOPE = np.float32(0.2)
_ONE = np.float32(1.0)
_BIG = np.float32(1e30)


def _dense_body(ent_ref, rel_ref, w1_ref, w2_ref, a1_ref, a2_ref,
                entw_ref, s1_ref, t2_ref, s2_ref):
    ew = jnp.dot(ent_ref[...], w1_ref[...], preferred_element_type=jnp.float32)
    entw_ref[...] = ew
    a1 = a1_ref[...]
    a2 = a2_ref[...]
    s1_ref[...] = jnp.dot(ew, a1[:_D, :], preferred_element_type=jnp.float32)
    t2_ref[...] = jnp.dot(ew, a2[_D:, :], preferred_element_type=jnp.float32)
    b2 = jnp.dot(w2_ref[...], a1[_D:, :], preferred_element_type=jnp.float32)
    s2_ref[...] = jnp.dot(rel_ref[...], b2, preferred_element_type=jnp.float32)


def _dense_call(entity_embs, relation_embs, w1, w2, a1, a2):
    blk = 2000
    grid = (_N // blk,)
    return pl.pallas_call(
        _dense_body,
        grid=grid,
        in_specs=[
            pl.BlockSpec((blk, _D), lambda i: (i, 0)),
            pl.BlockSpec((blk, _D), lambda i: (i, 0)),
            pl.BlockSpec((_D, _D), lambda i: (0, 0)),
            pl.BlockSpec((_D, _D), lambda i: (0, 0)),
            pl.BlockSpec((2 * _D, 1), lambda i: (0, 0)),
            pl.BlockSpec((2 * _D, 1), lambda i: (0, 0)),
        ],
        out_specs=[
            pl.BlockSpec((blk, _D), lambda i: (i, 0)),
            pl.BlockSpec((blk, 1), lambda i: (i, 0)),
            pl.BlockSpec((blk, 1), lambda i: (i, 0)),
            pl.BlockSpec((blk, 1), lambda i: (i, 0)),
        ],
        out_shape=[
            jax.ShapeDtypeStruct((_N, _D), jnp.float32),
            jax.ShapeDtypeStruct((_N, 1), jnp.float32),
            jax.ShapeDtypeStruct((_N, 1), jnp.float32),
            jax.ShapeDtypeStruct((_R, 1), jnp.float32),
        ],
    )(entity_embs, relation_embs, w1, w2, a1, a2)


def _sc_edge_body(tup_hbm, s1_hbm, s2_hbm, entw_hbm, edge_hbm,
                  tup_v, s1_v, s2_v, eidx_v, rows_v, out_v, sem):
    c = lax.axis_index("c")
    s = lax.axis_index("s")
    wid = s * _NC + c
    base_row = wid * _TUP_PER
    pltpu.sync_copy(tup_hbm.at[pl.ds(base_row * (_AR + 1), _TUP_PER * (_AR + 1))],
                    tup_v)
    pltpu.sync_copy(s1_hbm, s1_v)
    pltpu.sync_copy(s2_hbm, s2_v)

    def chunk(k, carry):
        rowids = (k * 16 + lax.iota(jnp.int32, 16)) * (_AR + 1)
        rel = plsc.load_gather(tup_v, [rowids])
        ridx = rel - 1
        ridx = jnp.where(ridx < 0, ridx + _R, ridx)
        s2g = plsc.load_gather(s2_v, [ridx])
        logits = []
        for ar in range(_AR):
            t = plsc.load_gather(tup_v, [rowids + (ar + 1)])
            eidx = t - 1
            eidx = jnp.where(eidx < 0, eidx + _N, eidx)
            eidx_v[pl.ds(ar * 16, 16)] = eidx
            lg = plsc.load_gather(s1_v, [eidx])
            lg = jnp.where(t > 0, lg, _NEG)
            x = lg + s2g
            logits.append(jnp.where(x > 0, x, _SLOPE * x))
        mx = logits[0]
        for ar in range(1, _AR):
            mx = jnp.maximum(mx, logits[ar])
        es = [jnp.exp(l - mx) for l in logits]
        tot = es[0]
        for ar in range(1, _AR):
            tot = tot + es[ar]
        inv = _ONE / tot
        att = [es[ar] * inv for ar in range(_AR)]
        pltpu.async_copy(entw_hbm.at[eidx_v], rows_v, sem).wait()
        for ti in range(16):
            for j in range(8):
                acc = att[0][ti] * rows_v[0 * 16 + ti, pl.ds(j * 16, 16)]
                for ar in range(1, _AR):
                    acc = acc + att[ar][ti] * rows_v[ar * 16 + ti, pl.ds(j * 16, 16)]
                out_v[ti, pl.ds(j * 16, 16)] = jnp.where(
                    acc > 0, acc, jnp.exp(acc) - _ONE)
        pltpu.sync_copy(out_v, edge_hbm.at[pl.ds(base_row + k * 16, 16), :])
        return carry

    lax.fori_loop(0, _CHUNKS, chunk, 0)


def _sc_edge_call(tup_pad, s1v, s2v, entity_w):
    mesh = plsc.VectorSubcoreMesh(core_axis_name="c", subcore_axis_name="s")
    fn = pl.kernel(
        _sc_edge_body,
        out_type=jax.ShapeDtypeStruct((_MP, _D), jnp.float32),
        mesh=mesh,
        compiler_params=pltpu.CompilerParams(needs_layout_passes=False),
        scratch_types=[
            pltpu.VMEM((_TUP_PER * (_AR + 1),), jnp.int32),
            pltpu.VMEM((_N,), jnp.float32),
            pltpu.VMEM((_R,), jnp.float32),
            pltpu.VMEM((_AR * 16,), jnp.int32),
            pltpu.VMEM((_AR * 16, _D), jnp.float32),
            pltpu.VMEM((16, _D), jnp.float32),
            pltpu.SemaphoreType.DMA,
        ],
    )
    return fn(tup_pad, s1v, s2v, entity_w)


def _t1_body(ed_ref, a2_ref, t1_ref):
    blk = ed_ref.shape[0]
    a2 = a2_ref[...]
    v = jnp.dot(ed_ref[...], a2[:_D, :], preferred_element_type=jnp.float32)
    row = (pl.program_id(0) * blk
           + lax.broadcasted_iota(jnp.int32, (blk, 1), 0))
    t1_ref[...] = jnp.where(row < _M, v, _BIG)


def _t1_call(edge_embs, a2):
    blk = 2048
    nb = _MP // blk
    return pl.pallas_call(
        _t1_body,
        grid=(nb,),
        in_specs=[
            pl.BlockSpec((blk, _D), lambda i: (i, 0)),
            pl.BlockSpec((2 * _D, 1), lambda i: (0, 0)),
        ],
        out_specs=pl.BlockSpec((blk, 1), lambda i: (i, 0)),
        out_shape=jax.ShapeDtypeStruct((_MP, 1), jnp.float32),
    )(edge_embs, a2)


def _sc_agg_body(nidx_hbm, eidx_hbm, t1_hbm, t2_hbm, edge_hbm, zero_hbm,
                 acc_out, rsum_out,
                 t1_v, t2_v, rsum_v, nidx_v, eidx_v, rows_v, acc_sh,
                 sem):
    c = lax.axis_index("c")
    s = lax.axis_index("s")
    wid = s * _NC + c

    @pl.when(s == 0)
    def _zero_acc():
        pltpu.sync_copy(zero_hbm, acc_sh)

    pltpu.sync_copy(t1_hbm, t1_v)
    pltpu.sync_copy(t2_hbm, t2_v)
    z16 = jnp.zeros((16,), jnp.float32)

    def zloop(i, carry):
        rsum_v[pl.ds(i * 16, 16)] = z16
        return carry

    lax.fori_loop(0, _N // 16, zloop, 0)
    plsc.subcore_barrier()

    base = wid * _NNZ_PER

    def blk(j, carry):
        off = base + j * _BLK_NNZ
        pltpu.sync_copy(nidx_hbm.at[pl.ds(off, _BLK_NNZ)], nidx_v.at[0])
        pltpu.sync_copy(eidx_hbm.at[pl.ds(off, _BLK_NNZ)], eidx_v.at[0])
        evecs = []
        for k in range(_BLK_NNZ // 16):
            ni = nidx_v[0, pl.ds(k * 16, 16)]
            ei = eidx_v[0, pl.ds(k * 16, 16)]
            x = plsc.load_gather(t1_v, [ei]) + plsc.load_gather(t2_v, [ni])
            x = jnp.where(x > 0, x, _SLOPE * x)
            e = jnp.exp(-x)
            evecs.append(e)
            plsc.addupdate_scatter(rsum_v, [ni], e)
        copies = [
            pltpu.async_copy(edge_hbm.at[eidx_v.at[0, pl.ds(q * 16, 16)]],
                             rows_v.at[pl.ds(q * 16, 16)], sem)
            for q in range(_BLK_NNZ // 16)
        ]
        for cp in copies:
            cp.wait()
        for b in range(_BLK_NNZ):
            w = evecs[b // 16][b % 16]
            for jj in range(_D // 16):
                sl = pl.ds(jj * 16, 16)
                rows_v[b, sl] = rows_v[b, sl] * w
        pltpu.sync_copy(rows_v, acc_sh.at[nidx_v.at[0]], add=True)
        return carry

    lax.fori_loop(0, _NBLK, blk, 0)
    plsc.subcore_barrier()

    @pl.when(s == 0)
    def _write_acc():
        pltpu.sync_copy(acc_sh, acc_out.at[c])

    pltpu.sync_copy(rsum_v, rsum_out.at[wid])


def _sc_agg_call(nidx_pad, eidx_pad, t1v, t2v, edge_embs, zeros_init):
    mesh = plsc.VectorSubcoreMesh(core_axis_name="c", subcore_axis_name="s")
    fn = pl.kernel(
        _sc_agg_body,
        out_type=[
            jax.ShapeDtypeStruct((_NC, _N, _D), jnp.float32),
            jax.ShapeDtypeStruct((_NW, _N), jnp.float32),
        ],
        mesh=mesh,
        compiler_params=pltpu.CompilerParams(needs_layout_passes=False),
        scratch_types=[
            pltpu.VMEM((_MP,), jnp.float32),
            pltpu.VMEM((_N,), jnp.float32),
            pltpu.VMEM((_N,), jnp.float32),
            pltpu.VMEM((1, _BLK_NNZ), jnp.int32),
            pltpu.VMEM((1, _BLK_NNZ), jnp.int32),
            pltpu.VMEM((_BLK_NNZ, _D), jnp.float32),
            pltpu.VMEM_SHARED((_N, _D), jnp.float32),
            pltpu.SemaphoreType.DMA,
        ],
    )
    return fn(nidx_pad, eidx_pad, t1v, t2v, edge_embs, zeros_init)


def _final_body(acc0_ref, acc1_ref, rsum_ref, out_ref):
    num = acc0_ref[...] + acc1_ref[...]
    rs = jnp.sum(rsum_ref[...], axis=1)
    x = num / rs[:, None]
    out_ref[...] = jnp.where(x > 0, x, jnp.exp(x) - _ONE)


def _final_call(acc0, acc1, rsum_t):
    blk = 1000
    return pl.pallas_call(
        _final_body,
        grid=(_N // blk,),
        in_specs=[
            pl.BlockSpec((blk, _D), lambda i: (i, 0)),
            pl.BlockSpec((blk, _D), lambda i: (i, 0)),
            pl.BlockSpec((blk, _NW), lambda i: (i, 0)),
        ],
        out_specs=pl.BlockSpec((blk, _D), lambda i: (i, 0)),
        out_shape=jax.ShapeDtypeStruct((_N, _D), jnp.float32),
    )(acc0, acc1, rsum_t)


def kernel(entity_embs, relation_embs, tuples, node_indices, edge_indices,
           W1, W2, a1, a2):
    entity_w, s1, t2, s2 = _dense_call(entity_embs, relation_embs, W1, W2, a1, a2)
    s1v = s1[:, 0]
    t2v = t2[:, 0]
    s2v = s2[:, 0]

    tup_pad = jnp.concatenate(
        [tuples.astype(jnp.int32),
         jnp.zeros((_MP - _M, _AR + 1), jnp.int32)], axis=0)
    edge_embs = _sc_edge_call(tup_pad.reshape(-1), s1v, s2v, entity_w)

    t1 = _t1_call(edge_embs, a2)
    t1v = t1[:, 0]

    nidx_pad = jnp.concatenate(
        [node_indices.astype(jnp.int32),
         jnp.zeros((_NNZP - _NNZ,), jnp.int32)])
    eidx_pad = jnp.concatenate(
        [edge_indices.astype(jnp.int32),
         jnp.full((_NNZP - _NNZ,), _M, jnp.int32)])
    zeros_init = jnp.zeros((_N, _D), jnp.float32)

    acc, rsum = _sc_agg_call(nidx_pad, eidx_pad, t1v, t2v, edge_embs,
                             zeros_init)
    return _final_call(acc[0], acc[1], rsum.T)


# speedup vs baseline: 5.4067x; 1.1562x over previous
import jax
import jax.numpy as jnp
import numpy as np
from jax import lax
from jax.experimental import pallas as pl
from jax.experimental.pallas import tpu as pltpu
from jax.experimental.pallas import tpu_sc as plsc

_N = 10000
_R = 10000
_M = 20000
_NNZ = 640000
_D = 128
_AR = 6

_NC = 2
_NS = 16
_NW = _NC * _NS

_MP = 20480
_TUP_PER = _MP // _NW
_CHUNKS = _TUP_PER // 16

_NNZP = 655360
_NNZ_PER = _NNZP // _NW
_BLK_NNZ = 64
_NBLK = _NNZ_PER // _BLK_NNZ

_NEG = np.float32(-9e15)
_SLOPE = np.float32(0.2)
_ONE = np.float32(1.0)
_BIG = np.float32(1e30)


def _dense_body(ent_ref, rel_ref, w1_ref, w2_ref, a1_ref, a2_ref,
                entw_ref, s1_ref, t2_ref, s2_ref):
    ew = jnp.dot(ent_ref[...], w1_ref[...], preferred_element_type=jnp.float32)
    entw_ref[...] = ew
    a1 = a1_ref[...]
    a2 = a2_ref[...]
    s1_ref[...] = jnp.dot(ew, a1[:_D, :], preferred_element_type=jnp.float32)
    t2_ref[...] = jnp.dot(ew, a2[_D:, :], preferred_element_type=jnp.float32)
    b2 = jnp.dot(w2_ref[...], a1[_D:, :], preferred_element_type=jnp.float32)
    s2_ref[...] = jnp.dot(rel_ref[...], b2, preferred_element_type=jnp.float32)


def _dense_call(entity_embs, relation_embs, w1, w2, a1, a2):
    blk = 2000
    grid = (_N // blk,)
    return pl.pallas_call(
        _dense_body,
        grid=grid,
        in_specs=[
            pl.BlockSpec((blk, _D), lambda i: (i, 0)),
            pl.BlockSpec((blk, _D), lambda i: (i, 0)),
            pl.BlockSpec((_D, _D), lambda i: (0, 0)),
            pl.BlockSpec((_D, _D), lambda i: (0, 0)),
            pl.BlockSpec((2 * _D, 1), lambda i: (0, 0)),
            pl.BlockSpec((2 * _D, 1), lambda i: (0, 0)),
        ],
        out_specs=[
            pl.BlockSpec((blk, _D), lambda i: (i, 0)),
            pl.BlockSpec((blk, 1), lambda i: (i, 0)),
            pl.BlockSpec((blk, 1), lambda i: (i, 0)),
            pl.BlockSpec((blk, 1), lambda i: (i, 0)),
        ],
        out_shape=[
            jax.ShapeDtypeStruct((_N, _D), jnp.float32),
            jax.ShapeDtypeStruct((_N, 1), jnp.float32),
            jax.ShapeDtypeStruct((_N, 1), jnp.float32),
            jax.ShapeDtypeStruct((_R, 1), jnp.float32),
        ],
    )(entity_embs, relation_embs, w1, w2, a1, a2)


def _sc_edge_body(tup_hbm, s1_hbm, s2_hbm, entw_hbm, edge_hbm,
                  tup_v, s1_v, s2_v, eidx_v, rows_v, out_v, sem):
    c = lax.axis_index("c")
    s = lax.axis_index("s")
    wid = s * _NC + c
    base_row = wid * _TUP_PER
    pltpu.sync_copy(tup_hbm.at[pl.ds(base_row * (_AR + 1), _TUP_PER * (_AR + 1))],
                    tup_v)
    pltpu.sync_copy(s1_hbm, s1_v)
    pltpu.sync_copy(s2_hbm, s2_v)

    def chunk(k, carry):
        rowids = (k * 16 + lax.iota(jnp.int32, 16)) * (_AR + 1)
        rel = plsc.load_gather(tup_v, [rowids])
        ridx = rel - 1
        ridx = jnp.where(ridx < 0, ridx + _R, ridx)
        s2g = plsc.load_gather(s2_v, [ridx])
        logits = []
        for ar in range(_AR):
            t = plsc.load_gather(tup_v, [rowids + (ar + 1)])
            eidx = t - 1
            eidx = jnp.where(eidx < 0, eidx + _N, eidx)
            eidx_v[pl.ds(ar * 16, 16)] = eidx
            lg = plsc.load_gather(s1_v, [eidx])
            lg = jnp.where(t > 0, lg, _NEG)
            x = lg + s2g
            logits.append(jnp.where(x > 0, x, _SLOPE * x))
        mx = logits[0]
        for ar in range(1, _AR):
            mx = jnp.maximum(mx, logits[ar])
        es = [jnp.exp(l - mx) for l in logits]
        tot = es[0]
        for ar in range(1, _AR):
            tot = tot + es[ar]
        inv = _ONE / tot
        att = [es[ar] * inv for ar in range(_AR)]
        pltpu.async_copy(entw_hbm.at[eidx_v], rows_v, sem).wait()
        for ti in range(16):
            for j in range(8):
                acc = att[0][ti] * rows_v[0 * 16 + ti, pl.ds(j * 16, 16)]
                for ar in range(1, _AR):
                    acc = acc + att[ar][ti] * rows_v[ar * 16 + ti, pl.ds(j * 16, 16)]
                out_v[ti, pl.ds(j * 16, 16)] = jnp.where(
                    acc > 0, acc, jnp.exp(acc) - _ONE)
        pltpu.sync_copy(out_v, edge_hbm.at[pl.ds(base_row + k * 16, 16), :])
        return carry

    lax.fori_loop(0, _CHUNKS, chunk, 0)


def _sc_edge_call(tup_pad, s1v, s2v, entity_w):
    mesh = plsc.VectorSubcoreMesh(core_axis_name="c", subcore_axis_name="s")
    fn = pl.kernel(
        _sc_edge_body,
        out_type=jax.ShapeDtypeStruct((_MP, _D), jnp.float32),
        mesh=mesh,
        compiler_params=pltpu.CompilerParams(needs_layout_passes=False),
        scratch_types=[
            pltpu.VMEM((_TUP_PER * (_AR + 1),), jnp.int32),
            pltpu.VMEM((_N,), jnp.float32),
            pltpu.VMEM((_R,), jnp.float32),
            pltpu.VMEM((_AR * 16,), jnp.int32),
            pltpu.VMEM((_AR * 16, _D), jnp.float32),
            pltpu.VMEM((16, _D), jnp.float32),
            pltpu.SemaphoreType.DMA,
        ],
    )
    return fn(tup_pad, s1v, s2v, entity_w)


def _t1_body(ed_ref, a2_ref, t1_ref):
    blk = ed_ref.shape[0]
    a2 = a2_ref[...]
    v = jnp.dot(ed_ref[...], a2[:_D, :], preferred_element_type=jnp.float32)
    row = (pl.program_id(0) * blk
           + lax.broadcasted_iota(jnp.int32, (blk, 1), 0))
    t1_ref[...] = jnp.where(row < _M, v, _BIG)


def _t1_call(edge_embs, a2):
    blk = 2048
    nb = _MP // blk
    return pl.pallas_call(
        _t1_body,
        grid=(nb,),
        in_specs=[
            pl.BlockSpec((blk, _D), lambda i: (i, 0)),
            pl.BlockSpec((2 * _D, 1), lambda i: (0, 0)),
        ],
        out_specs=pl.BlockSpec((blk, 1), lambda i: (i, 0)),
        out_shape=jax.ShapeDtypeStruct((_MP, 1), jnp.float32),
    )(edge_embs, a2)


def _sc_agg_body(nidx_hbm, eidx_hbm, t1_hbm, t2_hbm, edge_hbm, zero_hbm,
                 acc_out, rsum_out,
                 t1_v, t2_v, rsum_v, nidx_v, eidx_v, rows_v, acc_sh,
                 sem, sem2):
    c = lax.axis_index("c")
    s = lax.axis_index("s")
    wid = s * _NC + c

    @pl.when(s == 0)
    def _zero_acc():
        pltpu.sync_copy(zero_hbm, acc_sh)

    pltpu.sync_copy(t1_hbm, t1_v)
    pltpu.sync_copy(t2_hbm, t2_v)
    z16 = jnp.zeros((16,), jnp.float32)

    def zloop(i, carry):
        rsum_v[pl.ds(i * 16, 16)] = z16
        return carry

    lax.fori_loop(0, _N // 16, zloop, 0)
    plsc.subcore_barrier()

    base = wid * _NNZ_PER

    def idx_fetch(j, p):
        off = base + j * _BLK_NNZ
        c1 = pltpu.async_copy(nidx_hbm.at[pl.ds(off, _BLK_NNZ)],
                              nidx_v.at[p], sem2)
        c2 = pltpu.async_copy(eidx_hbm.at[pl.ds(off, _BLK_NNZ)],
                              eidx_v.at[p], sem2)
        return c1, c2

    w0a, w0b = idx_fetch(0, 0)
    w0a.wait()
    w0b.wait()

    def blk(j, carry):
        p = lax.rem(j, 2)
        jn = jnp.minimum(j + 1, _NBLK - 1)
        na, nb = idx_fetch(jn, 1 - p)
        evecs = []
        for k in range(_BLK_NNZ // 16):
            ni = nidx_v[p, pl.ds(k * 16, 16)]
            ei = eidx_v[p, pl.ds(k * 16, 16)]
            x = plsc.load_gather(t1_v, [ei]) + plsc.load_gather(t2_v, [ni])
            x = jnp.where(x > 0, x, _SLOPE * x)
            e = jnp.exp(-x)
            evecs.append(e)
            plsc.addupdate_scatter(rsum_v, [ni], e)
        pltpu.async_copy(edge_hbm.at[eidx_v.at[p]], rows_v, sem).wait()
        for b in range(_BLK_NNZ):
            w = evecs[b // 16][b % 16]
            for jj in range(_D // 16):
                sl = pl.ds(jj * 16, 16)
                rows_v[b, sl] = rows_v[b, sl] * w
        pltpu.sync_copy(rows_v, acc_sh.at[nidx_v.at[p]], add=True)
        na.wait()
        nb.wait()
        return carry

    lax.fori_loop(0, _NBLK, blk, 0)
    plsc.subcore_barrier()

    @pl.when(s == 0)
    def _write_acc():
        pltpu.sync_copy(acc_sh, acc_out.at[c])

    pltpu.sync_copy(rsum_v, rsum_out.at[wid])


def _sc_agg_call(nidx_pad, eidx_pad, t1v, t2v, edge_embs, zeros_init):
    mesh = plsc.VectorSubcoreMesh(core_axis_name="c", subcore_axis_name="s")
    fn = pl.kernel(
        _sc_agg_body,
        out_type=[
            jax.ShapeDtypeStruct((_NC, _N, _D), jnp.float32),
            jax.ShapeDtypeStruct((_NW, _N), jnp.float32),
        ],
        mesh=mesh,
        compiler_params=pltpu.CompilerParams(needs_layout_passes=False),
        scratch_types=[
            pltpu.VMEM((_MP,), jnp.float32),
            pltpu.VMEM((_N,), jnp.float32),
            pltpu.VMEM((_N,), jnp.float32),
            pltpu.VMEM((2, _BLK_NNZ), jnp.int32),
            pltpu.VMEM((2, _BLK_NNZ), jnp.int32),
            pltpu.VMEM((_BLK_NNZ, _D), jnp.float32),
            pltpu.VMEM_SHARED((_N, _D), jnp.float32),
            pltpu.SemaphoreType.DMA,
            pltpu.SemaphoreType.DMA,
        ],
    )
    return fn(nidx_pad, eidx_pad, t1v, t2v, edge_embs, zeros_init)


def _final_body(acc0_ref, acc1_ref, rsum_ref, out_ref):
    num = acc0_ref[...] + acc1_ref[...]
    rs = jnp.sum(rsum_ref[...], axis=1)
    x = num / rs[:, None]
    out_ref[...] = jnp.where(x > 0, x, jnp.exp(x) - _ONE)


def _final_call(acc0, acc1, rsum_t):
    blk = 1000
    return pl.pallas_call(
        _final_body,
        grid=(_N // blk,),
        in_specs=[
            pl.BlockSpec((blk, _D), lambda i: (i, 0)),
            pl.BlockSpec((blk, _D), lambda i: (i, 0)),
            pl.BlockSpec((blk, _NW), lambda i: (i, 0)),
        ],
        out_specs=pl.BlockSpec((blk, _D), lambda i: (i, 0)),
        out_shape=jax.ShapeDtypeStruct((_N, _D), jnp.float32),
    )(acc0, acc1, rsum_t)


def kernel(entity_embs, relation_embs, tuples, node_indices, edge_indices,
           W1, W2, a1, a2):
    entity_w, s1, t2, s2 = _dense_call(entity_embs, relation_embs, W1, W2, a1, a2)
    s1v = s1[:, 0]
    t2v = t2[:, 0]
    s2v = s2[:, 0]

    tup_pad = jnp.concatenate(
        [tuples.astype(jnp.int32),
         jnp.zeros((_MP - _M, _AR + 1), jnp.int32)], axis=0)
    edge_embs = _sc_edge_call(tup_pad.reshape(-1), s1v, s2v, entity_w)

    t1 = _t1_call(edge_embs, a2)
    t1v = t1[:, 0]

    nidx_pad = jnp.concatenate(
        [node_indices.astype(jnp.int32),
         jnp.zeros((_NNZP - _NNZ,), jnp.int32)])
    eidx_pad = jnp.concatenate(
        [edge_indices.astype(jnp.int32),
         jnp.full((_NNZP - _NNZ,), _M, jnp.int32)])
    zeros_init = jnp.zeros((_N, _D), jnp.float32)

    acc, rsum = _sc_agg_call(nidx_pad, eidx_pad, t1v, t2v, edge_embs,
                             zeros_init)
    return _final_call(acc[0], acc[1], rsum.T)
